# baseline (speedup 1.0000x reference)
import functools

import jax
import jax.numpy as jnp
from jax import lax
from jax.experimental import pallas as pl
from jax.experimental.pallas import tpu as pltpu
from jax.experimental.pallas import tpu_sc as plsc

IN_CH = 256
OUT_CH = 128
HEADS = 16
CPH = OUT_CH // HEADS
N_NODES = 10000
N_PAD = 10240
E_EDGES = 160000
NTILES = 32
CHUNK = 128
NCHUNK = 40
E_PER_TILE = CHUNK * NCHUNK
E_PADDED = E_PER_TILE * NTILES
ROWS_PER_TILE = N_PAD // 16
BLK = 512
NBLK = N_PAD // BLK
BLK_OUT = 1280
NBLK_OUT = E_EDGES // BLK_OUT
NBLK_PART = N_PAD // BLK_OUT
BND_BLK = N_NODES // BLK_OUT
HALF = OUT_CH // 2


def _tc_lin_body(x_ref, w_ref, m_ref, xw_ref, ssrc_ref, sdst_ref):
    xw = jnp.dot(x_ref[...], w_ref[...], preferred_element_type=jnp.float32)
    xw_ref[...] = xw
    s = jnp.dot(xw, m_ref[...], preferred_element_type=jnp.float32)
    ssrc_ref[...] = s[:, :HEADS]
    sdst_ref[...] = s[:, HEADS:]


def _tc_lin(x_pad, W, M):
    return pl.pallas_call(
        _tc_lin_body,
        grid=(NBLK,),
        in_specs=[
            pl.BlockSpec((BLK, IN_CH), lambda b: (b, 0)),
            pl.BlockSpec((IN_CH, OUT_CH), lambda b: (0, 0)),
            pl.BlockSpec((OUT_CH, 2 * HEADS), lambda b: (0, 0)),
        ],
        out_specs=[
            pl.BlockSpec((BLK, OUT_CH), lambda b: (b, 0)),
            pl.BlockSpec((BLK, HEADS), lambda b: (b, 0)),
            pl.BlockSpec((BLK, HEADS), lambda b: (b, 0)),
        ],
        out_shape=[
            jax.ShapeDtypeStruct((N_PAD, OUT_CH), jnp.float32),
            jax.ShapeDtypeStruct((N_PAD, HEADS), jnp.float32),
            jax.ShapeDtypeStruct((N_PAD, HEADS), jnp.float32),
        ],
    )(x_pad, W, M)


def _sc1_body(idxi_hbm, idxj_hbm, ssrc_hbm, sdst_hbm,
              s_out, cnt_out,
              idxi_v, idxj_v, a_v, b_v, e_v, ones_v, z_v,
              s_sh, cnt_sh, src_sh, dst_sh,
              sema, semb, ssem, osem):
    cid = lax.axis_index("c")
    sid = lax.axis_index("s")
    wid = sid * 2 + cid
    row0 = sid * ROWS_PER_TILE

    def _fill_ones(i, _):
        ones_v[i, :] = jnp.ones((16,), jnp.float32)
        return 0
    lax.fori_loop(0, CHUNK, _fill_ones, 0)

    def _fill_zero(i, _):
        z_v[i, :] = jnp.zeros((16,), jnp.float32)
        return 0
    lax.fori_loop(0, ROWS_PER_TILE, _fill_zero, 0)

    pltpu.sync_copy(z_v, s_sh.at[pl.ds(row0, ROWS_PER_TILE)])
    pltpu.sync_copy(z_v, cnt_sh.at[pl.ds(row0, ROWS_PER_TILE)])
    pltpu.sync_copy(ssrc_hbm.at[pl.ds(row0, ROWS_PER_TILE)],
                    src_sh.at[pl.ds(row0, ROWS_PER_TILE)])
    pltpu.sync_copy(sdst_hbm.at[pl.ds(row0, ROWS_PER_TILE)],
                    dst_sh.at[pl.ds(row0, ROWS_PER_TILE)])

    pltpu.sync_copy(idxi_hbm.at[pl.ds(wid * NCHUNK, NCHUNK)], idxi_v)
    pltpu.sync_copy(idxj_hbm.at[pl.ds(wid * NCHUNK, NCHUNK)], idxj_v)

    plsc.subcore_barrier()

    pltpu.async_copy(src_sh.at[idxi_v.at[0]], a_v.at[0], sema.at[0])
    pltpu.async_copy(dst_sh.at[idxj_v.at[0]], b_v.at[0], semb.at[0])

    def _chunk(c, _):
        buf = lax.rem(c, 2)
        nbuf = lax.rem(c + 1, 2)

        @pl.when(c + 1 < NCHUNK)
        def _prefetch():
            pltpu.async_copy(src_sh.at[idxi_v.at[c + 1]], a_v.at[nbuf],
                             sema.at[nbuf])
            pltpu.async_copy(dst_sh.at[idxj_v.at[c + 1]], b_v.at[nbuf],
                             semb.at[nbuf])

        pltpu.make_async_copy(src_sh.at[idxi_v.at[0]], a_v.at[buf],
                              sema.at[buf]).wait()
        pltpu.make_async_copy(dst_sh.at[idxj_v.at[0]], b_v.at[buf],
                              semb.at[buf]).wait()

        @pl.when(c >= 2)
        def _drain():
            pltpu.make_async_copy(e_v.at[buf], s_sh.at[idxi_v.at[0]],
                                  ssem.at[buf]).wait()

        @plsc.parallel_loop(0, CHUNK, unroll=8)
        def _edge(i):
            v = a_v[buf, i, :] + b_v[buf, i, :]
            v = jnp.where(v >= 0.0, v, 0.2 * v)
            e_v[buf, i, :] = jnp.exp(v)

        pltpu.async_copy(e_v.at[buf], s_sh.at[idxi_v.at[c]], ssem.at[buf],
                         add=True)
        pltpu.async_copy(ones_v, cnt_sh.at[idxj_v.at[c]], osem, add=True)
        return 0
    lax.fori_loop(0, NCHUNK, _chunk, 0)

    pltpu.make_async_copy(e_v.at[0], s_sh.at[idxi_v.at[0]],
                          ssem.at[0]).wait()
    pltpu.make_async_copy(e_v.at[1], s_sh.at[idxi_v.at[0]],
                          ssem.at[1]).wait()

    def _drain_ones(c, _):
        pltpu.make_async_copy(ones_v, cnt_sh.at[idxj_v.at[0]], osem).wait()
        return 0
    lax.fori_loop(0, NCHUNK, _drain_ones, 0)

    plsc.subcore_barrier()

    pltpu.sync_copy(s_sh.at[pl.ds(row0, ROWS_PER_TILE)],
                    s_out.at[cid, pl.ds(row0, ROWS_PER_TILE)])
    pltpu.sync_copy(cnt_sh.at[pl.ds(row0, ROWS_PER_TILE)],
                    cnt_out.at[cid, pl.ds(row0, ROWS_PER_TILE)])


def _sc1(idxi, idxj, ssrc, sdst):
    mesh = plsc.VectorSubcoreMesh(core_axis_name="c", subcore_axis_name="s")
    return pl.kernel(
        _sc1_body,
        out_type=[
            jax.ShapeDtypeStruct((2, N_PAD, HEADS), jnp.float32),
            jax.ShapeDtypeStruct((2, N_PAD, HEADS), jnp.float32),
        ],
        mesh=mesh,
        compiler_params=pltpu.CompilerParams(use_tc_tiling_on_sc=False),
        scratch_types=[
            pltpu.VMEM((NCHUNK, CHUNK), jnp.int32),
            pltpu.VMEM((NCHUNK, CHUNK), jnp.int32),
            pltpu.VMEM((2, CHUNK, HEADS), jnp.float32),
            pltpu.VMEM((2, CHUNK, HEADS), jnp.float32),
            pltpu.VMEM((2, CHUNK, HEADS), jnp.float32),
            pltpu.VMEM((CHUNK, HEADS), jnp.float32),
            pltpu.VMEM((ROWS_PER_TILE, HEADS), jnp.float32),
            pltpu.VMEM_SHARED((N_PAD, HEADS), jnp.float32),
            pltpu.VMEM_SHARED((N_PAD, HEADS), jnp.float32),
            pltpu.VMEM_SHARED((N_PAD, HEADS), jnp.float32),
            pltpu.VMEM_SHARED((N_PAD, HEADS), jnp.float32),
            pltpu.SemaphoreType.DMA((2,)),
            pltpu.SemaphoreType.DMA((2,)),
            pltpu.SemaphoreType.DMA((2,)),
            pltpu.SemaphoreType.DMA,
        ],
    )(idxi, idxj, ssrc, sdst)


MEMSET_ROWS = 32
YSLICE = 40


def _sc2_body(idxi_hbm, idxj_hbm, sp_hbm, cp_hbm, xw_hbm,
              out_part, dpart,
              idxi_v, idxj_v, ybuf, zbuf, xw_b, sp_b, cp_b, yout, dvec,
              y_sh, o_sh, gsem, ssem):
    cid = lax.axis_index("c")
    sid = lax.axis_index("s")
    wid = sid * 2 + cid
    row0 = sid * ROWS_PER_TILE

    def _fill_zero(i, _):
        def _lane(j, _):
            zbuf[i, pl.ds(j * 16, 16)] = jnp.zeros((16,), jnp.float32)
            return 0
        lax.fori_loop(0, HALF // 16, _lane, 0)
        return 0
    lax.fori_loop(0, MEMSET_ROWS, _fill_zero, 0)

    pltpu.sync_copy(idxi_hbm.at[pl.ds(wid * NCHUNK, NCHUNK)], idxi_v)
    pltpu.sync_copy(idxj_hbm.at[pl.ds(wid * NCHUNK, NCHUNK)], idxj_v)

    lane_lo = lax.iota(jnp.int32, 16) < 8

    for h in (0, 1):
        def _yslice(sl, dacc):
            r0 = row0 + sl * YSLICE
            pltpu.sync_copy(xw_hbm.at[pl.ds(r0, YSLICE)], xw_b)
            pltpu.sync_copy(sp_hbm.at[0, pl.ds(r0, YSLICE)], sp_b.at[0])
            pltpu.sync_copy(sp_hbm.at[1, pl.ds(r0, YSLICE)], sp_b.at[1])
            if h == 0:
                pltpu.sync_copy(cp_hbm.at[0, pl.ds(r0, YSLICE)], cp_b.at[0])
                pltpu.sync_copy(cp_hbm.at[1, pl.ds(r0, YSLICE)], cp_b.at[1])

            @plsc.parallel_loop(0, YSLICE, unroll=2, carry=dacc)
            def _row(r, d):
                srow = sp_b[0, r, :] + sp_b[1, r, :]
                if h == 0:
                    crow = cp_b[0, r, :] + cp_b[1, r, :]
                    d = d + crow * srow
                for j in range(HALF // 16):
                    s_a = srow[8 * h + 2 * j]
                    s_b = srow[8 * h + 2 * j + 1]
                    s_rep = jnp.where(lane_lo,
                                      jnp.full((16,), s_a, jnp.float32),
                                      jnp.full((16,), s_b, jnp.float32))
                    yout[r, pl.ds(16 * j, 16)] = (
                        s_rep * xw_b[r, pl.ds(64 * h + 16 * j, 16)])
                return d

            pltpu.sync_copy(yout, y_sh.at[pl.ds(r0, YSLICE)])
            return _row

        dacc = lax.fori_loop(0, ROWS_PER_TILE // YSLICE, _yslice,
                             jnp.zeros((16,), jnp.float32))
        if h == 0:
            dvec[0, :] = dacc
            pltpu.sync_copy(dvec, dpart.at[cid, pl.ds(sid, 1)])

        def _memset(k, _):
            pltpu.sync_copy(zbuf, o_sh.at[pl.ds(row0 + k * MEMSET_ROWS,
                                                MEMSET_ROWS)])
            return 0
        lax.fori_loop(0, ROWS_PER_TILE // MEMSET_ROWS, _memset, 0)

        plsc.subcore_barrier()

        pltpu.async_copy(y_sh.at[idxj_v.at[0]], ybuf.at[0], gsem.at[0])

        def _chunk(c, _):
            buf = lax.rem(c, 2)
            nbuf = lax.rem(c + 1, 2)

            @pl.when(c >= 1)
            def _drain_prev():
                pltpu.make_async_copy(ybuf.at[nbuf],
                                      o_sh.at[idxi_v.at[0]],
                                      ssem.at[nbuf]).wait()

            @pl.when(c + 1 < NCHUNK)
            def _prefetch():
                pltpu.async_copy(y_sh.at[idxj_v.at[c + 1]], ybuf.at[nbuf],
                                 gsem.at[nbuf])

            pltpu.make_async_copy(y_sh.at[idxj_v.at[c]], ybuf.at[buf],
                                  gsem.at[buf]).wait()
            pltpu.async_copy(ybuf.at[buf], o_sh.at[idxi_v.at[c]],
                             ssem.at[buf], add=True)
            return 0
        lax.fori_loop(0, NCHUNK, _chunk, 0)

        pltpu.make_async_copy(ybuf.at[1], o_sh.at[idxi_v.at[0]],
                              ssem.at[1]).wait()

        plsc.subcore_barrier()

        pltpu.sync_copy(o_sh.at[pl.ds(row0, ROWS_PER_TILE)],
                        out_part.at[h, cid, pl.ds(row0, ROWS_PER_TILE)])

        plsc.subcore_barrier()


def _sc2(idxi, idxj, s_part, cnt_part, xw):
    mesh = plsc.VectorSubcoreMesh(core_axis_name="c", subcore_axis_name="s")
    return pl.kernel(
        _sc2_body,
        out_type=[
            jax.ShapeDtypeStruct((2, 2, N_PAD, HALF), jnp.float32),
            jax.ShapeDtypeStruct((2, 16, HEADS), jnp.float32),
        ],
        mesh=mesh,
        compiler_params=pltpu.CompilerParams(use_tc_tiling_on_sc=False),
        scratch_types=[
            pltpu.VMEM((NCHUNK, CHUNK), jnp.int32),
            pltpu.VMEM((NCHUNK, CHUNK), jnp.int32),
            pltpu.VMEM((2, CHUNK, HALF), jnp.float32),
            pltpu.VMEM((MEMSET_ROWS, HALF), jnp.float32),
            pltpu.VMEM((YSLICE, OUT_CH), jnp.float32),
            pltpu.VMEM((2, YSLICE, HEADS), jnp.float32),
            pltpu.VMEM((2, YSLICE, HEADS), jnp.float32),
            pltpu.VMEM((YSLICE, HALF), jnp.float32),
            pltpu.VMEM((1, HEADS), jnp.float32),
            pltpu.VMEM_SHARED((N_PAD, HALF), jnp.float32),
            pltpu.VMEM_SHARED((N_PAD, HALF), jnp.float32),
            pltpu.SemaphoreType.DMA((2,)),
            pltpu.SemaphoreType.DMA((2,)),
        ],
    )(idxi, idxj, s_part, cnt_part, xw)


def _tc_out_body(op_ref, denom_ref, r_ref, bias_ref, out_ref):
    b = pl.program_id(0)
    denom = jnp.sum(denom_ref[0], axis=0, keepdims=True)
    inv = 1.0 / (denom + 1e-16)
    inv_rep = jnp.dot(inv, r_ref[...],
                      preferred_element_type=jnp.float32)
    bias_row = bias_ref[...]

    @pl.when(b < BND_BLK)
    def _real():
        acc = jnp.concatenate([op_ref[0, 0] + op_ref[0, 1],
                               op_ref[1, 0] + op_ref[1, 1]], axis=-1)
        out_ref[...] = acc * inv_rep + bias_row

    @pl.when(b == BND_BLK)
    def _boundary():
        acc = jnp.concatenate([op_ref[0, 0] + op_ref[0, 1],
                               op_ref[1, 0] + op_ref[1, 1]], axis=-1)
        row = b * BLK_OUT + lax.broadcasted_iota(jnp.int32,
                                                 (BLK_OUT, OUT_CH), 0)
        val = acc * inv_rep + bias_row
        pad = jnp.broadcast_to(bias_row, out_ref.shape)
        out_ref[...] = jnp.where(row < N_NODES, val, pad)

    @pl.when(b > BND_BLK)
    def _pad():
        out_ref[...] = jnp.broadcast_to(bias_row, out_ref.shape)


def _tc_out(out_part, denom, R, bias_row):
    return pl.pallas_call(
        _tc_out_body,
        grid=(NBLK_OUT,),
        in_specs=[
            pl.BlockSpec((2, 2, BLK_OUT, HALF),
                         lambda b: (0, 0, jnp.minimum(b, NBLK_PART - 1), 0)),
            pl.BlockSpec((2, 16, HEADS), lambda b: (0, 0, 0)),
            pl.BlockSpec((HEADS, OUT_CH), lambda b: (0, 0)),
            pl.BlockSpec((1, OUT_CH), lambda b: (0, 0)),
        ],
        out_specs=pl.BlockSpec((BLK_OUT, OUT_CH), lambda b: (b, 0)),
        out_shape=jax.ShapeDtypeStruct((E_EDGES, OUT_CH), jnp.float32),
    )(out_part, denom, R, bias_row)


@jax.jit
def kernel(x, edge_index, W, att, bias):
    x = x.astype(jnp.float32)
    W = W.astype(jnp.float32)
    att = att.astype(jnp.float32)
    bias = bias.astype(jnp.float32)

    eye = jnp.eye(HEADS, dtype=jnp.float32)
    m_src = (att[:, :CPH, None] * eye[:, None, :]).reshape(OUT_CH, HEADS)
    m_dst = (att[:, CPH:, None] * eye[:, None, :]).reshape(OUT_CH, HEADS)
    M = jnp.concatenate([m_src, m_dst], axis=1)
    R = jnp.kron(eye, jnp.ones((1, CPH), jnp.float32))

    x_pad = jnp.zeros((N_PAD, IN_CH), jnp.float32).at[:N_NODES].set(x)

    idx_i = edge_index[0].astype(jnp.int32)
    idx_j = edge_index[1].astype(jnp.int32)
    pad_n = E_PADDED - E_EDGES
    idx_i_p = jnp.concatenate(
        [idx_i, jnp.full((pad_n,), N_NODES, jnp.int32)]
    ).reshape(NTILES * NCHUNK, CHUNK)
    idx_j_p = jnp.concatenate(
        [idx_j, jnp.full((pad_n,), N_NODES + 1, jnp.int32)]
    ).reshape(NTILES * NCHUNK, CHUNK)

    xw, ssrc, sdst = _tc_lin(x_pad, W, M)
    s_part, cnt_part = _sc1(idx_i_p, idx_j_p, ssrc, sdst)
    out_part, dpart = _sc2(idx_i_p, idx_j_p, s_part, cnt_part, xw)
    out = _tc_out(out_part, dpart, R, bias.reshape(1, OUT_CH))
    return out


# speedup vs baseline: 1.1685x; 1.1685x over previous
import functools

import jax
import jax.numpy as jnp
from jax import lax
from jax.experimental import pallas as pl
from jax.experimental.pallas import tpu as pltpu
from jax.experimental.pallas import tpu_sc as plsc

IN_CH = 256
OUT_CH = 128
HEADS = 16
CPH = OUT_CH // HEADS
N_NODES = 10000
N_PAD = 10240
E_EDGES = 160000
NTILES = 32
CHUNK = 128
NCHUNK = 40
E_PER_TILE = CHUNK * NCHUNK
E_PADDED = E_PER_TILE * NTILES
ROWS_PER_TILE = N_PAD // 16
BLK = 512
NBLK = N_PAD // BLK
BLK_OUT = 1280
NBLK_OUT = E_EDGES // BLK_OUT
NBLK_PART = N_PAD // BLK_OUT
BND_BLK = N_NODES // BLK_OUT
HALF = OUT_CH // 2


def _tc_lin_body(x_ref, w_ref, m_ref, xw_ref, ssrc_ref, sdst_ref):
    xw = jnp.dot(x_ref[...], w_ref[...], preferred_element_type=jnp.float32)
    xw_ref[...] = xw
    s = jnp.dot(xw, m_ref[...], preferred_element_type=jnp.float32)
    ssrc_ref[...] = s[:, :HEADS]
    sdst_ref[...] = s[:, HEADS:]


def _tc_lin(x_pad, W, M):
    return pl.pallas_call(
        _tc_lin_body,
        grid=(NBLK,),
        in_specs=[
            pl.BlockSpec((BLK, IN_CH), lambda b: (b, 0)),
            pl.BlockSpec((IN_CH, OUT_CH), lambda b: (0, 0)),
            pl.BlockSpec((OUT_CH, 2 * HEADS), lambda b: (0, 0)),
        ],
        out_specs=[
            pl.BlockSpec((BLK, OUT_CH), lambda b: (b, 0)),
            pl.BlockSpec((BLK, HEADS), lambda b: (b, 0)),
            pl.BlockSpec((BLK, HEADS), lambda b: (b, 0)),
        ],
        out_shape=[
            jax.ShapeDtypeStruct((N_PAD, OUT_CH), jnp.float32),
            jax.ShapeDtypeStruct((N_PAD, HEADS), jnp.float32),
            jax.ShapeDtypeStruct((N_PAD, HEADS), jnp.float32),
        ],
    )(x_pad, W, M)


def _sc1_body(idxi_hbm, idxj_hbm, ssrc_hbm, sdst_hbm,
              s_out, cnt_out,
              idxi_v, idxj_v, a_v, b_v, e_v, ones_v, z_v,
              s_sh, cnt_sh, src_sh, dst_sh,
              sema, semb, ssem, osem):
    cid = lax.axis_index("c")
    sid = lax.axis_index("s")
    wid = sid * 2 + cid
    row0 = sid * ROWS_PER_TILE

    def _fill_ones(i, _):
        ones_v[i, :] = jnp.ones((16,), jnp.float32)
        return 0
    lax.fori_loop(0, CHUNK, _fill_ones, 0)

    def _fill_zero(i, _):
        z_v[i, :] = jnp.zeros((16,), jnp.float32)
        return 0
    lax.fori_loop(0, ROWS_PER_TILE, _fill_zero, 0)

    pltpu.sync_copy(z_v, s_sh.at[pl.ds(row0, ROWS_PER_TILE)])
    pltpu.sync_copy(z_v, cnt_sh.at[pl.ds(row0, ROWS_PER_TILE)])
    pltpu.sync_copy(ssrc_hbm.at[pl.ds(row0, ROWS_PER_TILE)],
                    src_sh.at[pl.ds(row0, ROWS_PER_TILE)])
    pltpu.sync_copy(sdst_hbm.at[pl.ds(row0, ROWS_PER_TILE)],
                    dst_sh.at[pl.ds(row0, ROWS_PER_TILE)])

    pltpu.sync_copy(idxi_hbm.at[pl.ds(wid * NCHUNK, NCHUNK)], idxi_v)
    pltpu.sync_copy(idxj_hbm.at[pl.ds(wid * NCHUNK, NCHUNK)], idxj_v)

    plsc.subcore_barrier()

    pltpu.async_copy(src_sh.at[idxi_v.at[0]], a_v.at[0], sema.at[0])
    pltpu.async_copy(dst_sh.at[idxj_v.at[0]], b_v.at[0], semb.at[0])

    def _chunk(c, _):
        buf = lax.rem(c, 2)
        nbuf = lax.rem(c + 1, 2)

        @pl.when(c + 1 < NCHUNK)
        def _prefetch():
            pltpu.async_copy(src_sh.at[idxi_v.at[c + 1]], a_v.at[nbuf],
                             sema.at[nbuf])
            pltpu.async_copy(dst_sh.at[idxj_v.at[c + 1]], b_v.at[nbuf],
                             semb.at[nbuf])

        pltpu.make_async_copy(src_sh.at[idxi_v.at[0]], a_v.at[buf],
                              sema.at[buf]).wait()
        pltpu.make_async_copy(dst_sh.at[idxj_v.at[0]], b_v.at[buf],
                              semb.at[buf]).wait()

        @pl.when(c >= 2)
        def _drain():
            pltpu.make_async_copy(e_v.at[buf], s_sh.at[idxi_v.at[0]],
                                  ssem.at[buf]).wait()

        @plsc.parallel_loop(0, CHUNK, unroll=8)
        def _edge(i):
            v = a_v[buf, i, :] + b_v[buf, i, :]
            v = jnp.where(v >= 0.0, v, 0.2 * v)
            e_v[buf, i, :] = jnp.exp(v)

        pltpu.async_copy(e_v.at[buf], s_sh.at[idxi_v.at[c]], ssem.at[buf],
                         add=True)
        pltpu.async_copy(ones_v, cnt_sh.at[idxj_v.at[c]], osem, add=True)
        return 0
    lax.fori_loop(0, NCHUNK, _chunk, 0)

    pltpu.make_async_copy(e_v.at[0], s_sh.at[idxi_v.at[0]],
                          ssem.at[0]).wait()
    pltpu.make_async_copy(e_v.at[1], s_sh.at[idxi_v.at[0]],
                          ssem.at[1]).wait()

    def _drain_ones(c, _):
        pltpu.make_async_copy(ones_v, cnt_sh.at[idxj_v.at[0]], osem).wait()
        return 0
    lax.fori_loop(0, NCHUNK, _drain_ones, 0)

    plsc.subcore_barrier()

    pltpu.sync_copy(s_sh.at[pl.ds(row0, ROWS_PER_TILE)],
                    s_out.at[cid, pl.ds(row0, ROWS_PER_TILE)])
    pltpu.sync_copy(cnt_sh.at[pl.ds(row0, ROWS_PER_TILE)],
                    cnt_out.at[cid, pl.ds(row0, ROWS_PER_TILE)])


def _sc1(idxi, idxj, ssrc, sdst):
    mesh = plsc.VectorSubcoreMesh(core_axis_name="c", subcore_axis_name="s")
    return pl.kernel(
        _sc1_body,
        out_type=[
            jax.ShapeDtypeStruct((2, N_PAD, HEADS), jnp.float32),
            jax.ShapeDtypeStruct((2, N_PAD, HEADS), jnp.float32),
        ],
        mesh=mesh,
        compiler_params=pltpu.CompilerParams(use_tc_tiling_on_sc=False),
        scratch_types=[
            pltpu.VMEM((NCHUNK, CHUNK), jnp.int32),
            pltpu.VMEM((NCHUNK, CHUNK), jnp.int32),
            pltpu.VMEM((2, CHUNK, HEADS), jnp.float32),
            pltpu.VMEM((2, CHUNK, HEADS), jnp.float32),
            pltpu.VMEM((2, CHUNK, HEADS), jnp.float32),
            pltpu.VMEM((CHUNK, HEADS), jnp.float32),
            pltpu.VMEM((ROWS_PER_TILE, HEADS), jnp.float32),
            pltpu.VMEM_SHARED((N_PAD, HEADS), jnp.float32),
            pltpu.VMEM_SHARED((N_PAD, HEADS), jnp.float32),
            pltpu.VMEM_SHARED((N_PAD, HEADS), jnp.float32),
            pltpu.VMEM_SHARED((N_PAD, HEADS), jnp.float32),
            pltpu.SemaphoreType.DMA((2,)),
            pltpu.SemaphoreType.DMA((2,)),
            pltpu.SemaphoreType.DMA((2,)),
            pltpu.SemaphoreType.DMA,
        ],
    )(idxi, idxj, ssrc, sdst)


def _tc_mid_body(s_ref, cnt_ref, xw_ref, r_ref, y0_ref, y1_ref, denom_ref):
    b = pl.program_id(0)
    s_tot = s_ref[0] + s_ref[1]
    cnt_tot = cnt_ref[0] + cnt_ref[1]
    y = jnp.dot(s_tot, r_ref[...],
                preferred_element_type=jnp.float32) * xw_ref[...]
    y0_ref[...] = y[:, :HALF]
    y1_ref[...] = y[:, HALF:]

    @pl.when(b == 0)
    def _init():
        denom_ref[...] = jnp.zeros_like(denom_ref)

    denom_ref[...] += jnp.sum(s_tot * cnt_tot, axis=0, keepdims=True)


def _tc_mid(s_part, cnt_part, xw, R):
    return pl.pallas_call(
        _tc_mid_body,
        grid=(NBLK,),
        in_specs=[
            pl.BlockSpec((2, BLK, HEADS), lambda b: (0, b, 0)),
            pl.BlockSpec((2, BLK, HEADS), lambda b: (0, b, 0)),
            pl.BlockSpec((BLK, OUT_CH), lambda b: (b, 0)),
            pl.BlockSpec((HEADS, OUT_CH), lambda b: (0, 0)),
        ],
        out_specs=[
            pl.BlockSpec((BLK, HALF), lambda b: (b, 0)),
            pl.BlockSpec((BLK, HALF), lambda b: (b, 0)),
            pl.BlockSpec((1, HEADS), lambda b: (0, 0)),
        ],
        out_shape=[
            jax.ShapeDtypeStruct((N_PAD, HALF), jnp.float32),
            jax.ShapeDtypeStruct((N_PAD, HALF), jnp.float32),
            jax.ShapeDtypeStruct((1, HEADS), jnp.float32),
        ],
    )(s_part, cnt_part, xw, R)


MEMSET_ROWS = 32


def _sc2_body(idxi_hbm, idxj_hbm, y0_hbm, y1_hbm,
              out_part,
              idxi_v, idxj_v, ybuf, zbuf, y_sh, o_sh, gsem, ssem):
    cid = lax.axis_index("c")
    sid = lax.axis_index("s")
    wid = sid * 2 + cid
    row0 = sid * ROWS_PER_TILE

    def _fill_zero(i, _):
        def _lane(j, _):
            zbuf[i, pl.ds(j * 16, 16)] = jnp.zeros((16,), jnp.float32)
            return 0
        lax.fori_loop(0, HALF // 16, _lane, 0)
        return 0
    lax.fori_loop(0, MEMSET_ROWS, _fill_zero, 0)

    pltpu.sync_copy(idxi_hbm.at[pl.ds(wid * NCHUNK, NCHUNK)], idxi_v)
    pltpu.sync_copy(idxj_hbm.at[pl.ds(wid * NCHUNK, NCHUNK)], idxj_v)

    for h, y_hbm in enumerate((y0_hbm, y1_hbm)):
        pltpu.sync_copy(y_hbm.at[pl.ds(row0, ROWS_PER_TILE)],
                        y_sh.at[pl.ds(row0, ROWS_PER_TILE)])

        def _memset(k, _):
            pltpu.sync_copy(zbuf, o_sh.at[pl.ds(row0 + k * MEMSET_ROWS,
                                                MEMSET_ROWS)])
            return 0
        lax.fori_loop(0, ROWS_PER_TILE // MEMSET_ROWS, _memset, 0)

        plsc.subcore_barrier()

        pltpu.async_copy(y_sh.at[idxj_v.at[0]], ybuf.at[0], gsem.at[0])

        def _chunk(c, _):
            buf = lax.rem(c, 2)
            nbuf = lax.rem(c + 1, 2)

            @pl.when(c >= 1)
            def _drain_prev():
                pltpu.make_async_copy(ybuf.at[nbuf],
                                      o_sh.at[idxi_v.at[0]],
                                      ssem.at[nbuf]).wait()

            @pl.when(c + 1 < NCHUNK)
            def _prefetch():
                pltpu.async_copy(y_sh.at[idxj_v.at[c + 1]], ybuf.at[nbuf],
                                 gsem.at[nbuf])

            pltpu.make_async_copy(y_sh.at[idxj_v.at[c]], ybuf.at[buf],
                                  gsem.at[buf]).wait()
            pltpu.async_copy(ybuf.at[buf], o_sh.at[idxi_v.at[c]],
                             ssem.at[buf], add=True)
            return 0
        lax.fori_loop(0, NCHUNK, _chunk, 0)

        pltpu.make_async_copy(ybuf.at[1], o_sh.at[idxi_v.at[0]],
                              ssem.at[1]).wait()

        plsc.subcore_barrier()

        pltpu.sync_copy(o_sh.at[pl.ds(row0, ROWS_PER_TILE)],
                        out_part.at[h, cid, pl.ds(row0, ROWS_PER_TILE)])

        plsc.subcore_barrier()


def _sc2(idxi, idxj, y0, y1):
    mesh = plsc.VectorSubcoreMesh(core_axis_name="c", subcore_axis_name="s")
    return pl.kernel(
        _sc2_body,
        out_type=jax.ShapeDtypeStruct((2, 2, N_PAD, HALF), jnp.float32),
        mesh=mesh,
        compiler_params=pltpu.CompilerParams(use_tc_tiling_on_sc=False),
        scratch_types=[
            pltpu.VMEM((NCHUNK, CHUNK), jnp.int32),
            pltpu.VMEM((NCHUNK, CHUNK), jnp.int32),
            pltpu.VMEM((2, CHUNK, HALF), jnp.float32),
            pltpu.VMEM((MEMSET_ROWS, HALF), jnp.float32),
            pltpu.VMEM_SHARED((N_PAD, HALF), jnp.float32),
            pltpu.VMEM_SHARED((N_PAD, HALF), jnp.float32),
            pltpu.SemaphoreType.DMA((2,)),
            pltpu.SemaphoreType.DMA((2,)),
        ],
    )(idxi, idxj, y0, y1)


def _tc_out_body(op_ref, denom_ref, r_ref, bias_ref, out_ref):
    b = pl.program_id(0)
    inv = 1.0 / (denom_ref[...] + 1e-16)
    inv_rep = jnp.dot(inv, r_ref[...],
                      preferred_element_type=jnp.float32)
    bias_row = bias_ref[...]

    @pl.when(b < BND_BLK)
    def _real():
        acc = jnp.concatenate([op_ref[0, 0] + op_ref[0, 1],
                               op_ref[1, 0] + op_ref[1, 1]], axis=-1)
        out_ref[...] = acc * inv_rep + bias_row

    @pl.when(b == BND_BLK)
    def _boundary():
        acc = jnp.concatenate([op_ref[0, 0] + op_ref[0, 1],
                               op_ref[1, 0] + op_ref[1, 1]], axis=-1)
        row = b * BLK_OUT + lax.broadcasted_iota(jnp.int32,
                                                 (BLK_OUT, OUT_CH), 0)
        val = acc * inv_rep + bias_row
        pad = jnp.broadcast_to(bias_row, out_ref.shape)
        out_ref[...] = jnp.where(row < N_NODES, val, pad)

    @pl.when(b > BND_BLK)
    def _pad():
        out_ref[...] = jnp.broadcast_to(bias_row, out_ref.shape)


def _tc_out(out_part, denom, R, bias_row):
    return pl.pallas_call(
        _tc_out_body,
        grid=(NBLK_OUT,),
        in_specs=[
            pl.BlockSpec((2, 2, BLK_OUT, HALF),
                         lambda b: (0, 0, jnp.minimum(b, NBLK_PART - 1), 0)),
            pl.BlockSpec((1, HEADS), lambda b: (0, 0)),
            pl.BlockSpec((HEADS, OUT_CH), lambda b: (0, 0)),
            pl.BlockSpec((1, OUT_CH), lambda b: (0, 0)),
        ],
        out_specs=pl.BlockSpec((BLK_OUT, OUT_CH), lambda b: (b, 0)),
        out_shape=jax.ShapeDtypeStruct((E_EDGES, OUT_CH), jnp.float32),
    )(out_part, denom, R, bias_row)


@jax.jit
def kernel(x, edge_index, W, att, bias):
    x = x.astype(jnp.float32)
    W = W.astype(jnp.float32)
    att = att.astype(jnp.float32)
    bias = bias.astype(jnp.float32)

    eye = jnp.eye(HEADS, dtype=jnp.float32)
    m_src = (att[:, :CPH, None] * eye[:, None, :]).reshape(OUT_CH, HEADS)
    m_dst = (att[:, CPH:, None] * eye[:, None, :]).reshape(OUT_CH, HEADS)
    M = jnp.concatenate([m_src, m_dst], axis=1)
    R = jnp.kron(eye, jnp.ones((1, CPH), jnp.float32))

    x_pad = jnp.zeros((N_PAD, IN_CH), jnp.float32).at[:N_NODES].set(x)

    idx_i = edge_index[0].astype(jnp.int32)
    idx_j = edge_index[1].astype(jnp.int32)
    pad_n = E_PADDED - E_EDGES
    idx_i_p = jnp.concatenate(
        [idx_i, jnp.full((pad_n,), N_NODES, jnp.int32)]
    ).reshape(NTILES * NCHUNK, CHUNK)
    idx_j_p = jnp.concatenate(
        [idx_j, jnp.full((pad_n,), N_NODES + 1, jnp.int32)]
    ).reshape(NTILES * NCHUNK, CHUNK)

    xw, ssrc, sdst = _tc_lin(x_pad, W, M)
    s_part, cnt_part = _sc1(idx_i_p, idx_j_p, ssrc, sdst)
    y0, y1, denom = _tc_mid(s_part, cnt_part, xw, R)
    out_part = _sc2(idx_i_p, idx_j_p, y0, y1)
    out = _tc_out(out_part, denom, R, bias.reshape(1, OUT_CH))
    return out


# speedup vs baseline: 1.1795x; 1.0095x over previous
import functools

import jax
import jax.numpy as jnp
from jax import lax
from jax.experimental import pallas as pl
from jax.experimental.pallas import tpu as pltpu
from jax.experimental.pallas import tpu_sc as plsc

IN_CH = 256
OUT_CH = 128
HEADS = 16
CPH = OUT_CH // HEADS
N_NODES = 10000
N_PAD = 10240
E_EDGES = 160000
NTILES = 32
CHUNK = 128
NCHUNK = 40
E_PER_TILE = CHUNK * NCHUNK
E_PADDED = E_PER_TILE * NTILES
ROWS_PER_TILE = N_PAD // 16
BLK = 512
NBLK = N_PAD // BLK
BLK_OUT = 1280
NBLK_OUT = E_EDGES // BLK_OUT
NBLK_PART = N_PAD // BLK_OUT
BND_BLK = N_NODES // BLK_OUT
HALF = OUT_CH // 2


def _tc_lin_body(x_ref, w_ref, m_ref, xw_ref, ssrc_ref, sdst_ref):
    xw = jnp.dot(x_ref[...], w_ref[...], preferred_element_type=jnp.float32)
    xw_ref[...] = xw
    s = jnp.dot(xw, m_ref[...], preferred_element_type=jnp.float32)
    ssrc_ref[...] = s[:, :HEADS]
    sdst_ref[...] = s[:, HEADS:]


def _tc_lin(x_pad, W, M):
    return pl.pallas_call(
        _tc_lin_body,
        grid=(NBLK,),
        in_specs=[
            pl.BlockSpec((BLK, IN_CH), lambda b: (b, 0)),
            pl.BlockSpec((IN_CH, OUT_CH), lambda b: (0, 0)),
            pl.BlockSpec((OUT_CH, 2 * HEADS), lambda b: (0, 0)),
        ],
        out_specs=[
            pl.BlockSpec((BLK, OUT_CH), lambda b: (b, 0)),
            pl.BlockSpec((BLK, HEADS), lambda b: (b, 0)),
            pl.BlockSpec((BLK, HEADS), lambda b: (b, 0)),
        ],
        out_shape=[
            jax.ShapeDtypeStruct((N_PAD, OUT_CH), jnp.float32),
            jax.ShapeDtypeStruct((N_PAD, HEADS), jnp.float32),
            jax.ShapeDtypeStruct((N_PAD, HEADS), jnp.float32),
        ],
    )(x_pad, W, M)


def _sc1_body(idxi_hbm, idxj_hbm, ssrc_hbm, sdst_hbm,
              s_out, cnt_out,
              idxi_v, idxj_v, a_v, b_v, e_v, ones_v, z_v,
              s_sh, cnt_sh, src_sh, dst_sh,
              sema, semb, ssem, osem):
    cid = lax.axis_index("c")
    sid = lax.axis_index("s")
    wid = sid * 2 + cid
    row0 = sid * ROWS_PER_TILE

    @plsc.parallel_loop(0, CHUNK, unroll=8)
    def _fill_ones(i):
        ones_v[i, :] = jnp.ones((16,), jnp.float32)

    @plsc.parallel_loop(0, ROWS_PER_TILE, unroll=8)
    def _fill_zero(i):
        z_v[i, :] = jnp.zeros((16,), jnp.float32)

    pltpu.sync_copy(z_v, s_sh.at[pl.ds(row0, ROWS_PER_TILE)])
    pltpu.sync_copy(z_v, cnt_sh.at[pl.ds(row0, ROWS_PER_TILE)])
    pltpu.sync_copy(ssrc_hbm.at[pl.ds(row0, ROWS_PER_TILE)],
                    src_sh.at[pl.ds(row0, ROWS_PER_TILE)])
    pltpu.sync_copy(sdst_hbm.at[pl.ds(row0, ROWS_PER_TILE)],
                    dst_sh.at[pl.ds(row0, ROWS_PER_TILE)])

    pltpu.sync_copy(idxi_hbm.at[pl.ds(wid * NCHUNK, NCHUNK)], idxi_v)
    pltpu.sync_copy(idxj_hbm.at[pl.ds(wid * NCHUNK, NCHUNK)], idxj_v)

    plsc.subcore_barrier()

    pltpu.async_copy(src_sh.at[idxi_v.at[0]], a_v.at[0], sema.at[0])
    pltpu.async_copy(dst_sh.at[idxj_v.at[0]], b_v.at[0], semb.at[0])

    def _chunk(c, _):
        buf = lax.rem(c, 2)
        nbuf = lax.rem(c + 1, 2)

        @pl.when(c + 1 < NCHUNK)
        def _prefetch():
            pltpu.async_copy(src_sh.at[idxi_v.at[c + 1]], a_v.at[nbuf],
                             sema.at[nbuf])
            pltpu.async_copy(dst_sh.at[idxj_v.at[c + 1]], b_v.at[nbuf],
                             semb.at[nbuf])

        pltpu.make_async_copy(src_sh.at[idxi_v.at[0]], a_v.at[buf],
                              sema.at[buf]).wait()
        pltpu.make_async_copy(dst_sh.at[idxj_v.at[0]], b_v.at[buf],
                              semb.at[buf]).wait()

        @pl.when(c >= 2)
        def _drain():
            pltpu.make_async_copy(e_v.at[buf], s_sh.at[idxi_v.at[0]],
                                  ssem.at[buf]).wait()

        @plsc.parallel_loop(0, CHUNK, unroll=8)
        def _edge(i):
            v = a_v[buf, i, :] + b_v[buf, i, :]
            v = jnp.where(v >= 0.0, v, 0.2 * v)
            e_v[buf, i, :] = jnp.exp(v)

        pltpu.async_copy(e_v.at[buf], s_sh.at[idxi_v.at[c]], ssem.at[buf],
                         add=True)
        pltpu.async_copy(ones_v, cnt_sh.at[idxj_v.at[c]], osem, add=True)
        return 0
    lax.fori_loop(0, NCHUNK, _chunk, 0)

    pltpu.make_async_copy(e_v.at[0], s_sh.at[idxi_v.at[0]],
                          ssem.at[0]).wait()
    pltpu.make_async_copy(e_v.at[1], s_sh.at[idxi_v.at[0]],
                          ssem.at[1]).wait()

    def _drain_ones(c, _):
        pltpu.make_async_copy(ones_v, cnt_sh.at[idxj_v.at[0]], osem).wait()
        return 0
    lax.fori_loop(0, NCHUNK, _drain_ones, 0)

    plsc.subcore_barrier()

    pltpu.sync_copy(s_sh.at[pl.ds(row0, ROWS_PER_TILE)],
                    s_out.at[cid, pl.ds(row0, ROWS_PER_TILE)])
    pltpu.sync_copy(cnt_sh.at[pl.ds(row0, ROWS_PER_TILE)],
                    cnt_out.at[cid, pl.ds(row0, ROWS_PER_TILE)])


def _sc1(idxi, idxj, ssrc, sdst):
    mesh = plsc.VectorSubcoreMesh(core_axis_name="c", subcore_axis_name="s")
    return pl.kernel(
        _sc1_body,
        out_type=[
            jax.ShapeDtypeStruct((2, N_PAD, HEADS), jnp.float32),
            jax.ShapeDtypeStruct((2, N_PAD, HEADS), jnp.float32),
        ],
        mesh=mesh,
        compiler_params=pltpu.CompilerParams(use_tc_tiling_on_sc=False),
        scratch_types=[
            pltpu.VMEM((NCHUNK, CHUNK), jnp.int32),
            pltpu.VMEM((NCHUNK, CHUNK), jnp.int32),
            pltpu.VMEM((2, CHUNK, HEADS), jnp.float32),
            pltpu.VMEM((2, CHUNK, HEADS), jnp.float32),
            pltpu.VMEM((2, CHUNK, HEADS), jnp.float32),
            pltpu.VMEM((CHUNK, HEADS), jnp.float32),
            pltpu.VMEM((ROWS_PER_TILE, HEADS), jnp.float32),
            pltpu.VMEM_SHARED((N_PAD, HEADS), jnp.float32),
            pltpu.VMEM_SHARED((N_PAD, HEADS), jnp.float32),
            pltpu.VMEM_SHARED((N_PAD, HEADS), jnp.float32),
            pltpu.VMEM_SHARED((N_PAD, HEADS), jnp.float32),
            pltpu.SemaphoreType.DMA((2,)),
            pltpu.SemaphoreType.DMA((2,)),
            pltpu.SemaphoreType.DMA((2,)),
            pltpu.SemaphoreType.DMA,
        ],
    )(idxi, idxj, ssrc, sdst)


def _tc_mid_body(s_ref, cnt_ref, xw_ref, r_ref, y0_ref, y1_ref, denom_ref):
    b = pl.program_id(0)
    s_tot = s_ref[0] + s_ref[1]
    cnt_tot = cnt_ref[0] + cnt_ref[1]
    y = jnp.dot(s_tot, r_ref[...],
                preferred_element_type=jnp.float32) * xw_ref[...]
    y0_ref[...] = y[:, :HALF]
    y1_ref[...] = y[:, HALF:]

    @pl.when(b == 0)
    def _init():
        denom_ref[...] = jnp.zeros_like(denom_ref)

    denom_ref[...] += jnp.sum(s_tot * cnt_tot, axis=0, keepdims=True)


def _tc_mid(s_part, cnt_part, xw, R):
    return pl.pallas_call(
        _tc_mid_body,
        grid=(NBLK,),
        in_specs=[
            pl.BlockSpec((2, BLK, HEADS), lambda b: (0, b, 0)),
            pl.BlockSpec((2, BLK, HEADS), lambda b: (0, b, 0)),
            pl.BlockSpec((BLK, OUT_CH), lambda b: (b, 0)),
            pl.BlockSpec((HEADS, OUT_CH), lambda b: (0, 0)),
        ],
        out_specs=[
            pl.BlockSpec((BLK, HALF), lambda b: (b, 0)),
            pl.BlockSpec((BLK, HALF), lambda b: (b, 0)),
            pl.BlockSpec((1, HEADS), lambda b: (0, 0)),
        ],
        out_shape=[
            jax.ShapeDtypeStruct((N_PAD, HALF), jnp.float32),
            jax.ShapeDtypeStruct((N_PAD, HALF), jnp.float32),
            jax.ShapeDtypeStruct((1, HEADS), jnp.float32),
        ],
    )(s_part, cnt_part, xw, R)


MEMSET_ROWS = 32


def _sc2_body(idxi_hbm, idxj_hbm, y0_hbm, y1_hbm,
              out_part,
              idxi_v, idxj_v, ybuf, zbuf, y_sh, o_sh, gsem, ssem):
    cid = lax.axis_index("c")
    sid = lax.axis_index("s")
    wid = sid * 2 + cid
    row0 = sid * ROWS_PER_TILE

    @plsc.parallel_loop(0, MEMSET_ROWS * (HALF // 16), unroll=8)
    def _fill_zero(i):
        zbuf[i // (HALF // 16), pl.ds((i % (HALF // 16)) * 16, 16)] = (
            jnp.zeros((16,), jnp.float32))

    pltpu.sync_copy(idxi_hbm.at[pl.ds(wid * NCHUNK, NCHUNK)], idxi_v)
    pltpu.sync_copy(idxj_hbm.at[pl.ds(wid * NCHUNK, NCHUNK)], idxj_v)

    for h, y_hbm in enumerate((y0_hbm, y1_hbm)):
        pltpu.sync_copy(y_hbm.at[pl.ds(row0, ROWS_PER_TILE)],
                        y_sh.at[pl.ds(row0, ROWS_PER_TILE)])

        def _memset(k, _):
            pltpu.sync_copy(zbuf, o_sh.at[pl.ds(row0 + k * MEMSET_ROWS,
                                                MEMSET_ROWS)])
            return 0
        lax.fori_loop(0, ROWS_PER_TILE // MEMSET_ROWS, _memset, 0)

        plsc.subcore_barrier()

        pltpu.async_copy(y_sh.at[idxj_v.at[0]], ybuf.at[0], gsem.at[0])

        def _chunk(c, _):
            buf = lax.rem(c, 2)
            nbuf = lax.rem(c + 1, 2)

            @pl.when(c >= 1)
            def _drain_prev():
                pltpu.make_async_copy(ybuf.at[nbuf],
                                      o_sh.at[idxi_v.at[0]],
                                      ssem.at[nbuf]).wait()

            @pl.when(c + 1 < NCHUNK)
            def _prefetch():
                pltpu.async_copy(y_sh.at[idxj_v.at[c + 1]], ybuf.at[nbuf],
                                 gsem.at[nbuf])

            pltpu.make_async_copy(y_sh.at[idxj_v.at[c]], ybuf.at[buf],
                                  gsem.at[buf]).wait()
            pltpu.async_copy(ybuf.at[buf], o_sh.at[idxi_v.at[c]],
                             ssem.at[buf], add=True)
            return 0
        lax.fori_loop(0, NCHUNK, _chunk, 0)

        pltpu.make_async_copy(ybuf.at[1], o_sh.at[idxi_v.at[0]],
                              ssem.at[1]).wait()

        plsc.subcore_barrier()

        pltpu.sync_copy(o_sh.at[pl.ds(row0, ROWS_PER_TILE)],
                        out_part.at[h, cid, pl.ds(row0, ROWS_PER_TILE)])

        plsc.subcore_barrier()


def _sc2(idxi, idxj, y0, y1):
    mesh = plsc.VectorSubcoreMesh(core_axis_name="c", subcore_axis_name="s")
    return pl.kernel(
        _sc2_body,
        out_type=jax.ShapeDtypeStruct((2, 2, N_PAD, HALF), jnp.float32),
        mesh=mesh,
        compiler_params=pltpu.CompilerParams(use_tc_tiling_on_sc=False),
        scratch_types=[
            pltpu.VMEM((NCHUNK, CHUNK), jnp.int32),
            pltpu.VMEM((NCHUNK, CHUNK), jnp.int32),
            pltpu.VMEM((2, CHUNK, HALF), jnp.float32),
            pltpu.VMEM((MEMSET_ROWS, HALF), jnp.float32),
            pltpu.VMEM_SHARED((N_PAD, HALF), jnp.float32),
            pltpu.VMEM_SHARED((N_PAD, HALF), jnp.float32),
            pltpu.SemaphoreType.DMA((2,)),
            pltpu.SemaphoreType.DMA((2,)),
        ],
    )(idxi, idxj, y0, y1)


def _tc_out_body(op_ref, denom_ref, r_ref, bias_ref, out_ref):
    b = pl.program_id(0)
    inv = 1.0 / (denom_ref[...] + 1e-16)
    inv_rep = jnp.dot(inv, r_ref[...],
                      preferred_element_type=jnp.float32)
    bias_row = bias_ref[...]

    @pl.when(b < BND_BLK)
    def _real():
        acc = jnp.concatenate([op_ref[0, 0] + op_ref[0, 1],
                               op_ref[1, 0] + op_ref[1, 1]], axis=-1)
        out_ref[...] = acc * inv_rep + bias_row

    @pl.when(b == BND_BLK)
    def _boundary():
        acc = jnp.concatenate([op_ref[0, 0] + op_ref[0, 1],
                               op_ref[1, 0] + op_ref[1, 1]], axis=-1)
        row = b * BLK_OUT + lax.broadcasted_iota(jnp.int32,
                                                 (BLK_OUT, OUT_CH), 0)
        val = acc * inv_rep + bias_row
        pad = jnp.broadcast_to(bias_row, out_ref.shape)
        out_ref[...] = jnp.where(row < N_NODES, val, pad)

    @pl.when(b > BND_BLK)
    def _pad():
        out_ref[...] = jnp.broadcast_to(bias_row, out_ref.shape)


def _tc_out(out_part, denom, R, bias_row):
    return pl.pallas_call(
        _tc_out_body,
        grid=(NBLK_OUT,),
        in_specs=[
            pl.BlockSpec((2, 2, BLK_OUT, HALF),
                         lambda b: (0, 0, jnp.minimum(b, NBLK_PART - 1), 0)),
            pl.BlockSpec((1, HEADS), lambda b: (0, 0)),
            pl.BlockSpec((HEADS, OUT_CH), lambda b: (0, 0)),
            pl.BlockSpec((1, OUT_CH), lambda b: (0, 0)),
        ],
        out_specs=pl.BlockSpec((BLK_OUT, OUT_CH), lambda b: (b, 0)),
        out_shape=jax.ShapeDtypeStruct((E_EDGES, OUT_CH), jnp.float32),
    )(out_part, denom, R, bias_row)


@jax.jit
def kernel(x, edge_index, W, att, bias):
    x = x.astype(jnp.float32)
    W = W.astype(jnp.float32)
    att = att.astype(jnp.float32)
    bias = bias.astype(jnp.float32)

    eye = jnp.eye(HEADS, dtype=jnp.float32)
    m_src = (att[:, :CPH, None] * eye[:, None, :]).reshape(OUT_CH, HEADS)
    m_dst = (att[:, CPH:, None] * eye[:, None, :]).reshape(OUT_CH, HEADS)
    M = jnp.concatenate([m_src, m_dst], axis=1)
    R = jnp.kron(eye, jnp.ones((1, CPH), jnp.float32))

    x_pad = jnp.zeros((N_PAD, IN_CH), jnp.float32).at[:N_NODES].set(x)

    idx_i = edge_index[0].astype(jnp.int32)
    idx_j = edge_index[1].astype(jnp.int32)
    pad_n = E_PADDED - E_EDGES
    idx_i_p = jnp.concatenate(
        [idx_i, jnp.full((pad_n,), N_NODES, jnp.int32)]
    ).reshape(NTILES * NCHUNK, CHUNK)
    idx_j_p = jnp.concatenate(
        [idx_j, jnp.full((pad_n,), N_NODES + 1, jnp.int32)]
    ).reshape(NTILES * NCHUNK, CHUNK)

    xw, ssrc, sdst = _tc_lin(x_pad, W, M)
    s_part, cnt_part = _sc1(idx_i_p, idx_j_p, ssrc, sdst)
    y0, y1, denom = _tc_mid(s_part, cnt_part, xw, R)
    out_part = _sc2(idx_i_p, idx_j_p, y0, y1)
    out = _tc_out(out_part, denom, R, bias.reshape(1, OUT_CH))
    return out
